# baseline (speedup 1.0000x reference)
import jax
import jax.numpy as jnp
from jax.experimental import pallas as pl
from jax.experimental.pallas import tpu as pltpu

N = 4096
D = 128
FC1 = 64
BM = 256
IB = N // BM


def _gcn_kernel(V_ref, adj_ref, w1_ref, w2_ref, w3_ref, bg_ref,
                fc0w_ref, fc0b_ref, fc1w_ref, fc1b_ref, out_ref,
                h1_ref, h2_ref, h3_ref, zsum_ref):
    i = pl.program_id(0)

    @pl.when(i == 0)
    def _compute_h():
        vblk = V_ref[:, :]
        h1_ref[:, :] = jnp.dot(
            vblk, w1_ref[:, :],
            preferred_element_type=jnp.float32).astype(jnp.bfloat16)
        h2_ref[:, :] = jnp.dot(
            vblk, w2_ref[:, :],
            preferred_element_type=jnp.float32).astype(jnp.bfloat16)
        h3_ref[:, :] = jnp.dot(
            vblk, w3_ref[:, :],
            preferred_element_type=jnp.float32).astype(jnp.bfloat16)

    a = adj_ref[:, :]
    m1 = (a == 1).astype(jnp.bfloat16)
    m2 = (a == 2).astype(jnp.bfloat16)
    m3 = (a == 3).astype(jnp.bfloat16)
    o = (jnp.dot(m1, h1_ref[:, :], preferred_element_type=jnp.float32)
         + jnp.dot(m2, h2_ref[:, :], preferred_element_type=jnp.float32)
         + jnp.dot(m3, h3_ref[:, :], preferred_element_type=jnp.float32))
    z = jnp.maximum(o + bg_ref[:, :], 0.0)
    part = jnp.sum(z, axis=0, keepdims=True)

    @pl.when(i == 0)
    def _init():
        zsum_ref[:, :] = part

    @pl.when(i > 0)
    def _accum():
        zsum_ref[:, :] += part

    @pl.when(i == IB - 1)
    def _head():
        zs = zsum_ref[:, :]
        h0 = jax.lax.dot_general(
            zs, fc0w_ref[:, :], (((1,), (1,)), ((), ())),
            preferred_element_type=jnp.float32) + fc0b_ref[:, :]
        h0 = jnp.maximum(h0, 0.0)
        y = jnp.sum(h0 * fc1w_ref[:, :] + fc1b_ref[:, :])
        out_ref[:, :] = jnp.full((1, 1), jax.nn.sigmoid(y), jnp.float32)


def kernel(V, adj, w1, w2, w3, bg, fc0_w, fc0_b, fc1_w, fc1_b):
    bg2 = bg.reshape(1, D)
    fc0b2 = fc0_b.reshape(1, FC1)
    fc1b2 = jnp.broadcast_to(fc1_b.reshape(1, 1) / FC1, (1, FC1))
    out = pl.pallas_call(
        _gcn_kernel,
        grid=(IB,),
        in_specs=[
            pl.BlockSpec((N, D), lambda i: (0, 0)),
            pl.BlockSpec((BM, N), lambda i: (i, 0)),
            pl.BlockSpec((D, D), lambda i: (0, 0)),
            pl.BlockSpec((D, D), lambda i: (0, 0)),
            pl.BlockSpec((D, D), lambda i: (0, 0)),
            pl.BlockSpec((1, D), lambda i: (0, 0)),
            pl.BlockSpec((FC1, D), lambda i: (0, 0)),
            pl.BlockSpec((1, FC1), lambda i: (0, 0)),
            pl.BlockSpec((1, FC1), lambda i: (0, 0)),
            pl.BlockSpec((1, FC1), lambda i: (0, 0)),
        ],
        out_specs=pl.BlockSpec((1, 1), lambda i: (0, 0)),
        out_shape=jax.ShapeDtypeStruct((1, 1), jnp.float32),
        scratch_shapes=[
            pltpu.VMEM((N, D), jnp.bfloat16),
            pltpu.VMEM((N, D), jnp.bfloat16),
            pltpu.VMEM((N, D), jnp.bfloat16),
            pltpu.VMEM((1, D), jnp.float32),
        ],
    )(V, adj, w1, w2, w3, bg2, fc0_w, fc0b2, fc1_w, fc1b2)
    return out.reshape(1)


# speedup vs baseline: 1.0427x; 1.0427x over previous
import jax
import jax.numpy as jnp
from jax.experimental import pallas as pl
from jax.experimental.pallas import tpu as pltpu

N = 4096
D = 128
FC1 = 64
BM = 1024
IB = N // BM


def _gcn_kernel(V_ref, adj_ref, w1_ref, w2_ref, w3_ref, bg_ref,
                fc0w_ref, fc0b_ref, fc1w_ref, fc1b_ref, out_ref,
                h1_ref, h2_ref, h3_ref, zsum_ref):
    i = pl.program_id(0)

    @pl.when(i == 0)
    def _compute_h():
        vblk = V_ref[:, :]
        h1_ref[:, :] = jnp.dot(
            vblk, w1_ref[:, :],
            preferred_element_type=jnp.float32).astype(jnp.bfloat16)
        h2_ref[:, :] = jnp.dot(
            vblk, w2_ref[:, :],
            preferred_element_type=jnp.float32).astype(jnp.bfloat16)
        h3_ref[:, :] = jnp.dot(
            vblk, w3_ref[:, :],
            preferred_element_type=jnp.float32).astype(jnp.bfloat16)

    a = adj_ref[:, :]
    m1 = (a == 1).astype(jnp.bfloat16)
    m2 = (a == 2).astype(jnp.bfloat16)
    m3 = (a == 3).astype(jnp.bfloat16)
    o = (jnp.dot(m1, h1_ref[:, :], preferred_element_type=jnp.float32)
         + jnp.dot(m2, h2_ref[:, :], preferred_element_type=jnp.float32)
         + jnp.dot(m3, h3_ref[:, :], preferred_element_type=jnp.float32))
    z = jnp.maximum(o + bg_ref[:, :], 0.0)
    part = jnp.sum(z, axis=0, keepdims=True)

    @pl.when(i == 0)
    def _init():
        zsum_ref[:, :] = part

    @pl.when(i > 0)
    def _accum():
        zsum_ref[:, :] += part

    @pl.when(i == IB - 1)
    def _head():
        zs = zsum_ref[:, :]
        h0 = jax.lax.dot_general(
            zs, fc0w_ref[:, :], (((1,), (1,)), ((), ())),
            preferred_element_type=jnp.float32) + fc0b_ref[:, :]
        h0 = jnp.maximum(h0, 0.0)
        y = jnp.sum(h0 * fc1w_ref[:, :] + fc1b_ref[:, :])
        out_ref[:, :] = jnp.full((1, 1), jax.nn.sigmoid(y), jnp.float32)


def kernel(V, adj, w1, w2, w3, bg, fc0_w, fc0_b, fc1_w, fc1_b):
    bg2 = bg.reshape(1, D)
    fc0b2 = fc0_b.reshape(1, FC1)
    fc1b2 = jnp.broadcast_to(fc1_b.reshape(1, 1) / FC1, (1, FC1))
    out = pl.pallas_call(
        _gcn_kernel,
        grid=(IB,),
        in_specs=[
            pl.BlockSpec((N, D), lambda i: (0, 0)),
            pl.BlockSpec((BM, N), lambda i: (i, 0)),
            pl.BlockSpec((D, D), lambda i: (0, 0)),
            pl.BlockSpec((D, D), lambda i: (0, 0)),
            pl.BlockSpec((D, D), lambda i: (0, 0)),
            pl.BlockSpec((1, D), lambda i: (0, 0)),
            pl.BlockSpec((FC1, D), lambda i: (0, 0)),
            pl.BlockSpec((1, FC1), lambda i: (0, 0)),
            pl.BlockSpec((1, FC1), lambda i: (0, 0)),
            pl.BlockSpec((1, FC1), lambda i: (0, 0)),
        ],
        out_specs=pl.BlockSpec((1, 1), lambda i: (0, 0)),
        out_shape=jax.ShapeDtypeStruct((1, 1), jnp.float32),
        scratch_shapes=[
            pltpu.VMEM((N, D), jnp.bfloat16),
            pltpu.VMEM((N, D), jnp.bfloat16),
            pltpu.VMEM((N, D), jnp.bfloat16),
            pltpu.VMEM((1, D), jnp.float32),
        ],
    )(V, adj, w1, w2, w3, bg2, fc0_w, fc0b2, fc1_w, fc1b2)
    return out.reshape(1)


# speedup vs baseline: 1.0870x; 1.0425x over previous
import jax
import jax.numpy as jnp
from jax.experimental import pallas as pl
from jax.experimental.pallas import tpu as pltpu

N = 4096
D = 128
FC1 = 64
BM = 512
IB = N // BM


def _gcn_kernel(V_ref, adj_ref, w1_ref, w2_ref, w3_ref, bg_ref,
                fc0w_ref, fc0b_ref, fc1w_ref, fc1b_ref, out_ref,
                h1_ref, h2_ref, h3_ref, zsum_ref):
    i = pl.program_id(0)

    @pl.when(i == 0)
    def _compute_h():
        vblk = V_ref[:, :]
        h1_ref[:, :] = jnp.dot(
            vblk, w1_ref[:, :],
            preferred_element_type=jnp.float32).astype(jnp.bfloat16)
        h2_ref[:, :] = jnp.dot(
            vblk, w2_ref[:, :],
            preferred_element_type=jnp.float32).astype(jnp.bfloat16)
        h3_ref[:, :] = jnp.dot(
            vblk, w3_ref[:, :],
            preferred_element_type=jnp.float32).astype(jnp.bfloat16)

    a = adj_ref[:, :]
    m1 = (a == 1).astype(jnp.bfloat16)
    m2 = (a == 2).astype(jnp.bfloat16)
    m3 = (a == 3).astype(jnp.bfloat16)
    o = (jnp.dot(m1, h1_ref[:, :], preferred_element_type=jnp.float32)
         + jnp.dot(m2, h2_ref[:, :], preferred_element_type=jnp.float32)
         + jnp.dot(m3, h3_ref[:, :], preferred_element_type=jnp.float32))
    z = jnp.maximum(o + bg_ref[:, :], 0.0)
    part = jnp.sum(z, axis=0, keepdims=True)

    @pl.when(i == 0)
    def _init():
        zsum_ref[:, :] = part

    @pl.when(i > 0)
    def _accum():
        zsum_ref[:, :] += part

    @pl.when(i == IB - 1)
    def _head():
        zs = zsum_ref[:, :]
        h0 = jax.lax.dot_general(
            zs, fc0w_ref[:, :], (((1,), (1,)), ((), ())),
            preferred_element_type=jnp.float32) + fc0b_ref[:, :]
        h0 = jnp.maximum(h0, 0.0)
        y = jnp.sum(h0 * fc1w_ref[:, :] + fc1b_ref[:, :])
        out_ref[:, :] = jnp.full((1, 1), jax.nn.sigmoid(y), jnp.float32)


def kernel(V, adj, w1, w2, w3, bg, fc0_w, fc0_b, fc1_w, fc1_b):
    bg2 = bg.reshape(1, D)
    fc0b2 = fc0_b.reshape(1, FC1)
    fc1b2 = jnp.broadcast_to(fc1_b.reshape(1, 1) / FC1, (1, FC1))
    out = pl.pallas_call(
        _gcn_kernel,
        grid=(IB,),
        in_specs=[
            pl.BlockSpec((N, D), lambda i: (0, 0)),
            pl.BlockSpec((BM, N), lambda i: (i, 0)),
            pl.BlockSpec((D, D), lambda i: (0, 0)),
            pl.BlockSpec((D, D), lambda i: (0, 0)),
            pl.BlockSpec((D, D), lambda i: (0, 0)),
            pl.BlockSpec((1, D), lambda i: (0, 0)),
            pl.BlockSpec((FC1, D), lambda i: (0, 0)),
            pl.BlockSpec((1, FC1), lambda i: (0, 0)),
            pl.BlockSpec((1, FC1), lambda i: (0, 0)),
            pl.BlockSpec((1, FC1), lambda i: (0, 0)),
        ],
        out_specs=pl.BlockSpec((1, 1), lambda i: (0, 0)),
        out_shape=jax.ShapeDtypeStruct((1, 1), jnp.float32),
        scratch_shapes=[
            pltpu.VMEM((N, D), jnp.bfloat16),
            pltpu.VMEM((N, D), jnp.bfloat16),
            pltpu.VMEM((N, D), jnp.bfloat16),
            pltpu.VMEM((1, D), jnp.float32),
        ],
    )(V, adj, w1, w2, w3, bg2, fc0_w, fc0b2, fc1_w, fc1b2)
    return out.reshape(1)
